# Initial kernel scaffold; baseline (speedup 1.0000x reference)
import jax
import jax.numpy as jnp
from jax.experimental import pallas as pl


def kernel(x, edge_index, W1, b1, W2, b2, W3, b3, Wc, bc):
    raise NotImplementedError("write your pallas kernel here")


# speedup vs baseline: 75.7941x; 75.7941x over previous
import functools

import jax
import jax.numpy as jnp
from jax import lax
from jax.experimental import pallas as pl
from jax.experimental.pallas import tpu as pltpu
from jax.experimental.pallas import tpu_sc as plsc

NC = 2
NS = 16
L = 16
NW = NC * NS

f32 = jnp.float32
i32 = jnp.int32


def _rsqrt16(x):
    xi = plsc.bitcast(x, i32)
    yi = jnp.int32(0x5F3759DF) - (xi >> 1)
    y = plsc.bitcast(yi, f32)
    for _ in range(3):
        y = y * (1.5 - 0.5 * x * y * y)
    return y


def _tanh16(x):
    e = jnp.exp(x * 2.0)
    return 1.0 - 2.0 / (e + 1.0)


def _edge_pass(g_v, acc_v, srcv, dstv, w, ept):
    jc = [jnp.full((L,), j, i32) for j in range(w)]

    @pl.loop(0, ept // L, unroll=5)
    def _(i):
        sl = pl.ds(i * L, L)
        s16 = srcv[sl]
        d16 = dstv[sl]
        dlo = d16 & 15
        dhi = d16 >> 4
        for j in range(w):
            m = plsc.load_gather(g_v, [jc[j], s16])
            plsc.addupdate_scatter(acc_v, [dlo, jc[j], dhi], m)


def _zero_acc(acc_v, w, cols):
    z = jnp.zeros((L,), f32)

    @pl.loop(0, cols // L)
    def _(i):
        sl = pl.ds(i * L, L)
        for r in range(NS):
            for j in range(w):
                acc_v[r, j, sl] = z


def _tile_ids():
    c = lax.axis_index("c")
    s = lax.axis_index("s")
    wid = s * NC + c
    return c, s, wid


def _half_cond(c, s):
    lo = (s < NS // 2) & (c == 0)
    hi = (s >= NS // 2) & (c == 1)
    return jnp.where(lo | hi, 1.0, 0.0)


def _make_deg_kernel(np_, ept):
    npt = np_ // NS
    npc = np_ // L
    cpt = npc // NS
    mesh = plsc.VectorSubcoreMesh(core_axis_name="c", subcore_axis_name="s")

    @functools.partial(
        pl.kernel,
        out_type=jax.ShapeDtypeStruct((NC, NS, L, cpt), f32),
        mesh=mesh,
        compiler_params=pltpu.CompilerParams(use_tc_tiling_on_sc=False,
                                             needs_layout_passes=False),
        scratch_types=[
            pltpu.VMEM((ept,), i32),
            pltpu.VMEM((L, npc), f32),
            pltpu.VMEM_SHARED((L, npc), f32),
        ],
    )
    def deg_kernel(dst_hbm, degp_out, dstv, cnt, cnt_sh):
        c, s, wid = _tile_ids()
        z = jnp.zeros((L,), f32)

        @pl.loop(0, npc // L)
        def _(i):
            sl = pl.ds(i * L, L)
            for r in range(L):
                cnt[r, sl] = z

        csl = pl.ds(s * cpt, cpt)
        pltpu.sync_copy(cnt.at[:, csl], cnt_sh.at[:, csl])
        plsc.subcore_barrier()

        pltpu.sync_copy(dst_hbm.at[pl.ds(wid * ept, ept)], dstv)
        ones = jnp.ones((L,), f32)

        @pl.loop(0, ept // L, unroll=5)
        def _(i):
            d16 = dstv[pl.ds(i * L, L)]
            plsc.addupdate_scatter(cnt, [d16 & 15, d16 >> 4], ones)

        pltpu.sync_copy(cnt, cnt_sh.at[lax.iota(i32, L)], add=True)
        plsc.subcore_barrier()
        pltpu.sync_copy(cnt_sh.at[:, csl], degp_out.at[c, s])

    return deg_kernel


def _make_l1_kernel(np_, ept):
    npt = np_ // NS
    npc = np_ // L
    cpt = npc // NS
    w = 4
    mesh = plsc.VectorSubcoreMesh(core_axis_name="c", subcore_axis_name="s")

    @functools.partial(
        pl.kernel,
        out_type=(
            jax.ShapeDtypeStruct((NC, NS, L, w, cpt), f32),
            jax.ShapeDtypeStruct((np_,), f32),
        ),
        mesh=mesh,
        compiler_params=pltpu.CompilerParams(use_tc_tiling_on_sc=False,
                                             needs_layout_passes=False),
        scratch_types=[
            pltpu.VMEM_SHARED((w, np_), f32),
            pltpu.VMEM_SHARED((L, w, npc), f32),
            pltpu.VMEM((w, np_), f32),
            pltpu.VMEM((L, w, npc), f32),
            pltpu.VMEM((ept,), i32),
            pltpu.VMEM((ept,), i32),
            pltpu.VMEM((npt, w), f32),
            pltpu.VMEM((L, cpt), f32),
            pltpu.VMEM((L, cpt), f32),
            pltpu.VMEM((npt,), f32),
            pltpu.VMEM((w, npt), f32),
            pltpu.VMEM((L, w, cpt), f32),
        ],
    )
    def l1_kernel(hp1_hbm, degp_hbm, src_hbm, dst_hbm,
                  accp_out, dinv_out,
                  g_sh, acc_sh, g_v, acc_v, srcv, dstv,
                  hp_v, d0, d1, dinv_v, gbuf, ibuf):
        c, s, wid = _tile_ids()
        base_n = s * npt
        csl = pl.ds(s * cpt, cpt)
        fcond = _half_cond(c, s)

        pltpu.sync_copy(hp1_hbm.at[pl.ds(base_n, npt), :], hp_v)
        pltpu.sync_copy(degp_hbm.at[0, s], d0)
        pltpu.sync_copy(degp_hbm.at[1, s], d1)
        pltpu.sync_copy(src_hbm.at[pl.ds(wid * ept, ept)], srcv)
        pltpu.sync_copy(dst_hbm.at[pl.ds(wid * ept, ept)], dstv)

        jc = [jnp.full((L,), j, i32) for j in range(w)]
        lanes = lax.iota(i32, L)
        zi = jnp.zeros((L,), i32)

        @pl.loop(0, npt // L)
        def _(i):
            sl = pl.ds(i * L, L)
            rows = lanes + i * L
            ifull = zi + i
            deg = (plsc.load_gather(d0, [lanes, ifull])
                   + plsc.load_gather(d1, [lanes, ifull]) + 1.0)
            dv = _rsqrt16(deg)
            dinv_v[sl] = dv
            for j in range(w):
                hcol = plsc.load_gather(hp_v, [rows, jc[j]])
                gj = dv * hcol
                gbuf[j, sl] = gj
                plsc.store_scatter(ibuf, [lanes, jc[j], ifull], gj * fcond)

        for j in range(w):
            pltpu.sync_copy(gbuf.at[j], g_sh.at[j, pl.ds(base_n, npt)])
        pltpu.sync_copy(ibuf, acc_sh.at[:, :, csl])

        @pl.when(c == 0)
        def _():
            pltpu.sync_copy(dinv_v, dinv_out.at[pl.ds(base_n, npt)])

        plsc.subcore_barrier()
        pltpu.sync_copy(g_sh, g_v)
        _zero_acc(acc_v, w, npc)
        _edge_pass(g_v, acc_v, srcv, dstv, w, ept)
        pltpu.sync_copy(acc_v, acc_sh.at[lax.iota(i32, L)], add=True)
        plsc.subcore_barrier()
        pltpu.sync_copy(acc_sh.at[:, :, csl], accp_out.at[c, s])

    return l1_kernel


def _make_mid_kernel(np_, ept, w_in, w_out, b_off, w_off):
    npt = np_ // NS
    npc = np_ // L
    cpt = npc // NS
    mesh = plsc.VectorSubcoreMesh(core_axis_name="c", subcore_axis_name="s")

    @functools.partial(
        pl.kernel,
        out_type=jax.ShapeDtypeStruct((NC, NS, L, w_out, cpt), f32),
        mesh=mesh,
        compiler_params=pltpu.CompilerParams(use_tc_tiling_on_sc=False,
                                             needs_layout_passes=False),
        scratch_types=[
            pltpu.VMEM_SHARED((w_out, np_), f32),
            pltpu.VMEM_SHARED((L, w_out, npc), f32),
            pltpu.VMEM((w_out, np_), f32),
            pltpu.VMEM((L, w_out, npc), f32),
            pltpu.VMEM((ept,), i32),
            pltpu.VMEM((ept,), i32),
            pltpu.VMEM((L, w_in, cpt), f32),
            pltpu.VMEM((L, w_in, cpt), f32),
            pltpu.VMEM((npt,), f32),
            pltpu.VMEM((w_out, npt), f32),
            pltpu.VMEM((L, w_out, cpt), f32),
            pltpu.VMEM((64,), f32),
        ],
    )
    def mid_kernel(accp_in, dinv_hbm, pbuf_hbm, src_hbm, dst_hbm,
                   accp_out,
                   g_sh, acc_sh, g_v, acc_v, srcv, dstv,
                   a0, a1, dinv_v, gbuf, ibuf, pv):
        c, s, wid = _tile_ids()
        base_n = s * npt
        csl = pl.ds(s * cpt, cpt)
        fcond = _half_cond(c, s)

        pltpu.sync_copy(pbuf_hbm, pv)
        pltpu.sync_copy(accp_in.at[0, s], a0)
        pltpu.sync_copy(accp_in.at[1, s], a1)
        pltpu.sync_copy(dinv_hbm.at[pl.ds(base_n, npt)], dinv_v)
        pltpu.sync_copy(src_hbm.at[pl.ds(wid * ept, ept)], srcv)
        pltpu.sync_copy(dst_hbm.at[pl.ds(wid * ept, ept)], dstv)

        jci = [jnp.full((L,), j, i32) for j in range(w_in)]
        jco = [jnp.full((L,), j, i32) for j in range(w_out)]
        lanes = lax.iota(i32, L)
        zi = jnp.zeros((L,), i32)

        pvecs = [pv[pl.ds(16 * t, L)] for t in range(4)]

        def _p(off):
            return pvecs[off // L][off % L]

        bias = [_p(b_off + j) for j in range(w_in)]
        wmat = [[_p(w_off + j * w_out + k) for k in range(w_out)]
                for j in range(w_in)]

        @pl.loop(0, npt // L)
        def _(i):
            sl = pl.ds(i * L, L)
            ifull = zi + i
            dv = dinv_v[sl]
            h = []
            for j in range(w_in):
                pre = (plsc.load_gather(a0, [lanes, jci[j], ifull])
                       + plsc.load_gather(a1, [lanes, jci[j], ifull]))
                h.append(_tanh16(dv * pre + bias[j]))
            for k in range(w_out):
                acc = h[0] * wmat[0][k]
                for j in range(1, w_in):
                    acc = acc + h[j] * wmat[j][k]
                gk = dv * acc
                gbuf[k, sl] = gk
                plsc.store_scatter(ibuf, [lanes, jco[k], ifull], gk * fcond)

        for k in range(w_out):
            pltpu.sync_copy(gbuf.at[k], g_sh.at[k, pl.ds(base_n, npt)])
        pltpu.sync_copy(ibuf, acc_sh.at[:, :, csl])

        plsc.subcore_barrier()
        pltpu.sync_copy(g_sh, g_v)
        _zero_acc(acc_v, w_out, npc)
        _edge_pass(g_v, acc_v, srcv, dstv, w_out, ept)
        pltpu.sync_copy(acc_v, acc_sh.at[lax.iota(i32, L)], add=True)
        plsc.subcore_barrier()
        pltpu.sync_copy(acc_sh.at[:, :, csl], accp_out.at[c, s])

    return mid_kernel


def _mm_body(x_ref, w_ref, o_ref):
    o_ref[...] = jnp.dot(x_ref[...], w_ref[...],
                         preferred_element_type=f32,
                         precision=lax.Precision.HIGHEST)


def _fin_body(a0_ref, a1_ref, dinv_ref, b3_ref, wc_ref, bc_ref,
              lo_ref, h_ref):
    dv = dinv_ref[...]
    hs = []
    for j in range(2):
        pre = dv * (a0_ref[j] + a1_ref[j]) + b3_ref[j]
        hj = jnp.tanh(pre)
        h_ref[j] = hj
        hs.append(hj)
    for k in range(7):
        lo_ref[k] = hs[0] * wc_ref[0, k] + hs[1] * wc_ref[1, k] + bc_ref[k]


def kernel(x, edge_index, W1, b1, W2, b2, W3, b3, Wc, bc):
    n, df = x.shape
    e = edge_index.shape[1]
    np_ = ((n + NS * L - 1) // (NS * L)) * NS * L
    ep = ((e + NW * L - 1) // (NW * L)) * NW * L
    ept = ep // NW
    npc = np_ // L
    rows2d = np_ // 128

    xp = jnp.pad(x, ((0, np_ - n), (0, 0)))
    src = edge_index[0]
    dst = edge_index[1]
    if ep != e:
        fill = jnp.full((ep - e,), n, i32)
        src = jnp.concatenate([src, fill])
        dst = jnp.concatenate([dst, fill])
    pbuf = jnp.concatenate([
        b1, W2.reshape(-1), b2, W3.reshape(-1), b3, Wc.reshape(-1), bc,
    ])
    pbuf = jnp.pad(pbuf, (0, 64 - pbuf.shape[0]))

    blk = 1280
    hp1 = pl.pallas_call(
        _mm_body,
        grid=(np_ // blk,),
        in_specs=[
            pl.BlockSpec((blk, df), lambda i: (i, 0)),
            pl.BlockSpec((df, 4), lambda i: (0, 0)),
        ],
        out_specs=pl.BlockSpec((blk, 4), lambda i: (i, 0)),
        out_shape=jax.ShapeDtypeStruct((np_, 4), f32),
    )(xp, W1)

    degp = _make_deg_kernel(np_, ept)(dst)
    accp1, dinv = _make_l1_kernel(np_, ept)(hp1, degp, src, dst)
    accp2 = _make_mid_kernel(np_, ept, 4, 4, 0, 4)(accp1, dinv, pbuf, src, dst)
    accp3 = _make_mid_kernel(np_, ept, 4, 2, 20, 24)(accp2, dinv, pbuf, src, dst)

    a_lin = accp3.transpose(0, 3, 1, 4, 2).reshape(NC, 2, np_)
    a0 = a_lin[0].reshape(2, rows2d, 128)
    a1 = a_lin[1].reshape(2, rows2d, 128)
    dinv2d = dinv.reshape(rows2d, 128)
    lo, h = pl.pallas_call(
        _fin_body,
        in_specs=[
            pl.BlockSpec(memory_space=pltpu.VMEM),
            pl.BlockSpec(memory_space=pltpu.VMEM),
            pl.BlockSpec(memory_space=pltpu.VMEM),
            pl.BlockSpec(memory_space=pltpu.SMEM),
            pl.BlockSpec(memory_space=pltpu.SMEM),
            pl.BlockSpec(memory_space=pltpu.SMEM),
        ],
        out_shape=(
            jax.ShapeDtypeStruct((7, rows2d, 128), f32),
            jax.ShapeDtypeStruct((2, rows2d, 128), f32),
        ),
    )(a0, a1, dinv2d, b3, Wc, bc)

    logits = jnp.moveaxis(lo, 0, -1).reshape(np_, 7)[:n]
    hout = jnp.moveaxis(h, 0, -1).reshape(np_, 2)[:n]
    return (logits, hout)
